# baseline (speedup 1.0000x reference)
import functools

import jax
import jax.numpy as jnp
from jax import lax
from jax.experimental import pallas as pl
from jax.experimental.pallas import tpu as pltpu
from jax.experimental.pallas import tpu_sc as plsc

_CH = 128
_ZB = 128
_NSUB = 16
_BLK = 512
_G = 64


@functools.lru_cache(maxsize=None)
def _make_propagate(n_pad, fh, nchunk, npass):
  rows_per_tile = n_pad // _NSUB
  ntab = 2 * npass
  mesh = plsc.VectorSubcoreMesh(core_axis_name="c", subcore_axis_name="s")

  @functools.partial(
      pl.kernel,
      out_type=[jax.ShapeDtypeStruct((n_pad, fh), jnp.float32)] * ntab,
      mesh=mesh,
      scratch_types=[
          pltpu.VMEM((nchunk, _CH), jnp.int32),
          pltpu.VMEM((nchunk, _CH), jnp.int32),
          pltpu.VMEM((4, _CH, fh), jnp.float32),
          pltpu.VMEM((_ZB, fh), jnp.float32),
          pltpu.VMEM_SHARED((n_pad, fh), jnp.float32),
          pltpu.SemaphoreType.DMA,
          pltpu.SemaphoreType.DMA,
      ],
      compiler_params=pltpu.CompilerParams(use_tc_tiling_on_sc=False),
  )
  def prop(*refs):
    tables = refs[:ntab]
    srcb, dstb, zrows = refs[ntab:ntab + 3]
    outs = refs[ntab + 3:2 * ntab + 3]
    src_v, dst_v, rows_v, zero_v, acc_sh, sem_g, sem_s = refs[2 * ntab + 3:]

    c = lax.axis_index("c")
    s = lax.axis_index("s")
    base = s * rows_per_tile
    sl = pl.ds(base, rows_per_tile)

    pltpu.sync_copy(zrows, zero_v)
    pltpu.sync_copy(srcb.at[s], src_v)
    pltpu.sync_copy(dstb.at[s], dst_v)

    def run(tbl):
      def gather(j):
        pltpu.async_copy(tbl.at[src_v.at[j]], rows_v.at[j % 4], sem_g)

      def wait_gather(j):
        pltpu.make_async_copy(tbl.at[src_v.at[j]], rows_v.at[j % 4],
                              sem_g).wait()

      def scatter(j):
        pltpu.async_copy(rows_v.at[j % 4], acc_sh.at[dst_v.at[j]], sem_s,
                         add=True)

      def wait_scatter(j):
        pltpu.make_async_copy(rows_v.at[j % 4], acc_sh.at[dst_v.at[j]],
                              sem_s).wait()

      for jj in range(min(2, nchunk)):
        gather(jj)

      @pl.loop(0, nchunk)
      def _(j):
        @pl.when(j >= 2)
        def _():
          wait_scatter(j - 2)

        @pl.when(j + 2 < nchunk)
        def _():
          gather(j + 2)

        wait_gather(j)
        scatter(j)

      @pl.loop(max(nchunk - 2, 0), nchunk)
      def _(j):
        wait_scatter(j)

    for p in range(npass):
      @pl.loop(0, rows_per_tile // _ZB)
      def _(i):
        pltpu.sync_copy(zero_v, acc_sh.at[pl.ds(base + i * _ZB, _ZB)])

      plsc.subcore_barrier()

      @pl.when(c == 0)
      def _():
        run(tables[p])

      @pl.when(c == 1)
      def _():
        run(tables[npass + p])

      plsc.subcore_barrier()

      @pl.when(c == 0)
      def _():
        pltpu.sync_copy(acc_sh.at[sl], outs[p].at[sl])

      @pl.when(c == 1)
      def _():
        pltpu.sync_copy(acc_sh.at[sl], outs[npass + p].at[sl])

  return prop


_IBW = 8


@functools.lru_cache(maxsize=None)
def _make_propagate_wide(n_pad, nchunk):
  fh = 128
  chw = 96
  ibw = _IBW
  rows_per_tile = n_pad // _NSUB
  zb = 16
  nblk = nchunk // ibw
  mesh = plsc.VectorSubcoreMesh(core_axis_name="c", subcore_axis_name="s")

  @functools.partial(
      pl.kernel,
      out_type=[jax.ShapeDtypeStruct((n_pad, fh), jnp.float32)] * 2,
      mesh=mesh,
      scratch_types=[
          pltpu.VMEM((2, ibw, chw), jnp.int32),
          pltpu.VMEM((2, ibw, chw), jnp.int32),
          pltpu.VMEM((3, chw, fh), jnp.float32),
          pltpu.VMEM((zb, fh), jnp.float32),
          pltpu.VMEM_SHARED((n_pad, fh), jnp.float32),
          pltpu.SemaphoreType.DMA,
          pltpu.SemaphoreType.DMA,
          pltpu.SemaphoreType.DMA,
      ],
      compiler_params=pltpu.CompilerParams(use_tc_tiling_on_sc=False),
  )
  def prop(t0, t1, srcb, dstb, zrows, out0, out1,
           src_v, dst_v, rows_v, zero_v, acc_sh, sem_g, sem_s, sem_i):
    c = lax.axis_index("c")
    s = lax.axis_index("s")
    base = s * rows_per_tile
    sl = pl.ds(base, rows_per_tile)

    pltpu.sync_copy(zrows, zero_v)

    @pl.loop(0, rows_per_tile // zb)
    def _(i):
      pltpu.sync_copy(zero_v, acc_sh.at[pl.ds(base + i * zb, zb)])

    def stage_idx(b):
      pltpu.async_copy(srcb.at[s, pl.ds(b * ibw, ibw)], src_v.at[b % 2],
                       sem_i)
      pltpu.async_copy(dstb.at[s, pl.ds(b * ibw, ibw)], dst_v.at[b % 2],
                       sem_i)

    def wait_idx(b):
      pltpu.make_async_copy(srcb.at[s, pl.ds(b * ibw, ibw)],
                            src_v.at[b % 2], sem_i).wait()
      pltpu.make_async_copy(dstb.at[s, pl.ds(b * ibw, ibw)],
                            dst_v.at[b % 2], sem_i).wait()

    plsc.subcore_barrier()

    def run(tbl):
      def gather(b, jj, buf):
        pltpu.async_copy(tbl.at[src_v.at[b % 2, jj]], rows_v.at[buf], sem_g)

      def wait_gather(b, jj, buf):
        pltpu.make_async_copy(tbl.at[src_v.at[b % 2, jj]], rows_v.at[buf],
                              sem_g).wait()

      def scatter(b, jj, buf):
        pltpu.async_copy(rows_v.at[buf], acc_sh.at[dst_v.at[b % 2, jj]],
                         sem_s, add=True)

      def wait_scatter(b, jj, buf):
        pltpu.make_async_copy(rows_v.at[buf], acc_sh.at[dst_v.at[b % 2, jj]],
                              sem_s).wait()

      stage_idx(0)
      wait_idx(0)
      if nblk > 1:
        stage_idx(1)
      gather(0, 0, 0)

      @pl.loop(0, nblk)
      def _(b):
        @pl.loop(0, ibw)
        def _(jj):
          j = b * ibw + jj

          @pl.when(j >= 2)
          def _():
            wait_scatter(b, jj, (j + 1) % 3)

          @pl.when((jj == 1) & (b >= 1) & (b + 1 < nblk))
          def _():
            stage_idx(b + 1)

          @pl.when(jj + 1 < ibw)
          def _():
            gather(b, jj + 1, (j + 1) % 3)

          wait_gather(b, jj, j % 3)
          scatter(b, jj, j % 3)

        @pl.when(b + 1 < nblk)
        def _():
          wait_idx(b + 1)
          gather(b + 1, 0, (b * ibw + ibw) % 3)

      @pl.loop(nblk * ibw - 2, nblk * ibw)
      def _(j):
        pltpu.make_async_copy(rows_v.at[j % 3],
                              acc_sh.at[dst_v.at[(nblk - 1) % 2, 0]],
                              sem_s).wait()

    @pl.when(c == 0)
    def _():
      run(t0)

    @pl.when(c == 1)
    def _():
      run(t1)

    plsc.subcore_barrier()

    @pl.when(c == 0)
    def _():
      pltpu.sync_copy(acc_sh.at[sl], out0.at[sl])

    @pl.when(c == 1)
    def _():
      pltpu.sync_copy(acc_sh.at[sl], out1.at[sl])

  return prop


def _dot(a, b):
  return jnp.dot(a, b, precision=lax.Precision.HIGHEST,
                 preferred_element_type=jnp.float32)


def _row_spec(cols):
  return pl.BlockSpec((_BLK, cols), lambda i: (i, 0))


def _full_spec(r, c):
  return pl.BlockSpec((r, c), lambda i: (0, 0))


def _split(u, nout):
  p = u.shape[1] // nout
  return [u[:, k * p:(k + 1) * p] for k in range(nout)]


@functools.lru_cache(maxsize=None)
def _make_tc0(n_pad, f_in, nout):
  nb = n_pad // _BLK
  q = f_in // nout

  def body(x_ref, deg_ref, dinv_ref, *xt_refs):
    cnt = deg_ref[:, 0:1]
    dinv = lax.rsqrt(cnt + 1.0)
    dinv_ref[...] = jnp.broadcast_to(dinv, (_BLK, 128))
    xt = x_ref[...] * dinv
    for r, piece in zip(xt_refs, _split(xt, nout)):
      r[...] = piece

  return pl.pallas_call(
      body,
      grid=(nb,),
      in_specs=[_row_spec(f_in), _row_spec(16)],
      out_specs=[_row_spec(128)] + [_row_spec(q)] * nout,
      out_shape=[jax.ShapeDtypeStruct((n_pad, 128), jnp.float32)]
      + [jax.ShapeDtypeStruct((n_pad, q), jnp.float32)] * nout,
  )


@functools.lru_cache(maxsize=None)
def _make_tc01(n_pad, f_in, h_dim, nin, nout):
  nb = n_pad // _BLK
  qi = f_in // nin
  qo = h_dim // nout

  def body(*refs):
    it = iter(refs)
    accs = [next(it) for _ in range(nin)]
    xts = [next(it) for _ in range(nin)]
    dinv_ref, b_ref, w0_ref, w1_ref = next(it), next(it), next(it), next(it)
    outs = list(it)

    dinv = dinv_ref[:, 0:1]
    p = jnp.concatenate([dinv * (a[...] + t[...]) for a, t in zip(accs, xts)],
                        axis=1)
    h1 = jnp.maximum(_dot(p, w0_ref[...]) + b_ref[...], 0.0)
    for r, piece in zip(outs[:nout], _split(h1, nout)):
      r[...] = piece
    u1 = _dot(h1, w1_ref[...]) * dinv
    for r, piece in zip(outs[nout:], _split(u1, nout)):
      r[...] = piece

  in_specs = [_row_spec(qi)] * (2 * nin)
  in_specs += [_row_spec(128), _full_spec(1, h_dim),
               _full_spec(f_in, h_dim), _full_spec(h_dim, h_dim)]
  out_specs = [_row_spec(qo)] * (2 * nout)
  out_shape = [jax.ShapeDtypeStruct((n_pad, qo), jnp.float32)] * (2 * nout)

  return pl.pallas_call(
      body, grid=(nb,), in_specs=in_specs, out_specs=out_specs,
      out_shape=out_shape)


@functools.lru_cache(maxsize=None)
def _make_tc_layer(n_pad, h_dim, w_cols, nin, nout, with_resid, emit_h):
  nb = n_pad // _BLK
  qi = h_dim // nin
  qo = w_cols // nout

  def body(*refs):
    it = iter(refs)
    accs = [next(it) for _ in range(nin)]
    us = [next(it) for _ in range(nin)]
    rs = [next(it) for _ in range(nin)] if with_resid else None
    dinv_ref, b_ref, w_ref = next(it), next(it), next(it)
    outs = list(it)

    dinv = dinv_ref[:, 0:1]
    b = b_ref[...]
    hs = []
    for k in range(nin):
      hk = jnp.maximum(
          dinv * (accs[k][...] + us[k][...]) + b[:, k * qi:(k + 1) * qi], 0.0)
      if with_resid:
        hk = hk + rs[k][...]
      hs.append(hk)
    if emit_h:
      for r, hk in zip(outs[:nin], hs):
        r[...] = hk
      outs = outs[nin:]
    h = jnp.concatenate(hs, axis=1)
    u = _dot(h, w_ref[...]) * dinv
    for r, piece in zip(outs, _split(u, nout)):
      r[...] = piece

  in_specs = [_row_spec(qi)] * (nin * (3 if with_resid else 2))
  in_specs += [_row_spec(128), _full_spec(1, h_dim), _full_spec(h_dim, w_cols)]

  out_specs = []
  out_shape = []
  if emit_h:
    out_specs += [_row_spec(qi)] * nin
    out_shape += [jax.ShapeDtypeStruct((n_pad, qi), jnp.float32)] * nin
  out_specs += [_row_spec(qo)] * nout
  out_shape += [jax.ShapeDtypeStruct((n_pad, qo), jnp.float32)] * nout

  return pl.pallas_call(
      body, grid=(nb,), in_specs=in_specs, out_specs=out_specs,
      out_shape=out_shape)


@functools.lru_cache(maxsize=None)
def _make_tc_pool(n_pad, c_dim, nin):
  nb = n_pad // _BLK
  qi = c_dim // nin

  def body(*refs):
    it = iter(refs)
    accs = [next(it) for _ in range(nin)]
    us = [next(it) for _ in range(nin)]
    dinv_ref, b_ref, bt_ref, out_ref, pooled = (next(it), next(it), next(it),
                                                next(it), next(it))
    i = pl.program_id(0)

    @pl.when(i == 0)
    def _():
      pooled[...] = jnp.zeros((_G, 128), jnp.float32)

    dinv = dinv_ref[:, 0:1]
    h = dinv * jnp.concatenate([a[...] + u[...] for a, u in zip(accs, us)],
                               axis=1) + b_ref[...]
    xc = jnp.concatenate([h, jnp.ones((_BLK, 128 - c_dim), jnp.float32)],
                         axis=1)
    onehot = (bt_ref[...] == lax.broadcasted_iota(jnp.int32, (_BLK, _G), 1)
              ).astype(jnp.float32)
    pooled[...] += lax.dot_general(
        onehot, xc, (((0,), (0,)), ((), ())),
        precision=lax.Precision.HIGHEST,
        preferred_element_type=jnp.float32)

    @pl.when(i == nb - 1)
    def _():
      p = pooled[...]
      mean = p[:, :c_dim] / jnp.maximum(p[:, c_dim:c_dim + 1], 1.0)
      z = mean - jnp.max(mean, axis=1, keepdims=True)
      lse = jnp.log(jnp.sum(jnp.exp(z), axis=1, keepdims=True))
      out_ref[...] = z - lse

  return pl.pallas_call(
      body,
      grid=(nb,),
      in_specs=[_row_spec(qi)] * (2 * nin)
      + [_row_spec(128), _full_spec(1, c_dim),
         pl.BlockSpec((_BLK, 1), lambda i: (i, 0))],
      out_specs=pl.BlockSpec((_G, c_dim), lambda i: (0, 0)),
      out_shape=jax.ShapeDtypeStruct((_G, c_dim), jnp.float32),
      scratch_shapes=[pltpu.VMEM((_G, 128), jnp.float32)],
  )


def kernel(x, edge_index, batch, W0, b0, W1, b1, W2, b2, W3, b3):
  n, f_in = x.shape
  e = edge_index.shape[1]
  h_dim = W0.shape[1]
  c_dim = W3.shape[1]

  n_pad = -(-n // 2048) * 2048
  nchunk = -(-e // (_NSUB * _CH))
  ep = _NSUB * nchunk * _CH
  chw = 96
  nchunk_w = -(-(-(-e // (_NSUB * chw))) // _IBW) * _IBW
  ep_w = _NSUB * nchunk_w * chw

  src32 = edge_index[0].astype(jnp.int32)
  dst32 = edge_index[1].astype(jnp.int32)
  pad_idx = jnp.full((ep - e,), n, jnp.int32)
  srcb = jnp.concatenate([src32, pad_idx]).reshape(_NSUB, nchunk, _CH)
  dstb = jnp.concatenate([dst32, pad_idx]).reshape(_NSUB, nchunk, _CH)
  pad_w = jnp.full((ep_w - e,), n, jnp.int32)
  srcw = jnp.concatenate([src32, pad_w]).reshape(_NSUB, nchunk_w, chw)
  dstw = jnp.concatenate([dst32, pad_w]).reshape(_NSUB, nchunk_w, chw)

  xp = jnp.pad(x, ((0, n_pad - n), (0, 0)))
  bt = jnp.pad(batch.astype(jnp.int32), (0, n_pad - n),
               constant_values=_G).reshape(n_pad, 1)

  ones16 = jnp.ones((n_pad, 16), jnp.float32)
  z16 = jnp.zeros((_ZB, 16), jnp.float32)
  zh = jnp.zeros((_ZB, h_dim // 4), jnp.float32)
  zc = jnp.zeros((_ZB, c_dim // 2), jnp.float32)

  prop_deg = _make_propagate(n_pad, 16, nchunk, 1)
  prop_x = _make_propagate(n_pad, f_in // 2, nchunk, 1)
  prop_h = _make_propagate_wide(n_pad, nchunk_w)
  prop_c = _make_propagate(n_pad, c_dim // 2, nchunk, 1)

  tc0 = _make_tc0(n_pad, f_in, 2)
  tc1 = _make_tc01(n_pad, f_in, h_dim, 2, 2)
  tc2 = _make_tc_layer(n_pad, h_dim, h_dim, 2, 2, True, False)
  tc3 = _make_tc_layer(n_pad, h_dim, c_dim, 2, 2, True, False)
  tc4 = _make_tc_pool(n_pad, c_dim, 2)

  b0r = b0.reshape(1, -1)
  b1r = b1.reshape(1, -1)
  b2r = b2.reshape(1, -1)
  b3r = b3.reshape(1, -1)

  zx = jnp.zeros((_ZB, f_in // 2), jnp.float32)
  zw = jnp.zeros((16, 128), jnp.float32)
  deg = prop_deg(ones16, ones16, srcb, dstb, z16)[0]
  dinv_b, *xt = tc0(xp, deg)

  ax = prop_x(*xt, srcb, dstb, zx)
  h1a, h1b, *u1 = tc1(*ax, *xt, dinv_b, b0r, W0, W1)

  a1 = prop_h(*u1, srcw, dstw, zw)
  u2 = tc2(*a1, *u1, h1a, h1b, dinv_b, b1r, W2)

  a2 = prop_h(*u2, srcw, dstw, zw)
  u3 = tc3(*a2, *u2, h1a, h1b, dinv_b, b2r, W3)

  a3 = prop_c(*u3, srcb, dstb, zc)
  return tc4(*a3, *u3, dinv_b, b3r, bt)


# speedup vs baseline: 2.2042x; 2.2042x over previous
import functools

import jax
import jax.numpy as jnp
from jax import lax
from jax.experimental import pallas as pl
from jax.experimental.pallas import tpu as pltpu
from jax.experimental.pallas import tpu_sc as plsc

_CH = 128
_ZB = 64
_NSUB = 16
_BLK = 512
_G = 64


@functools.lru_cache(maxsize=None)
def _make_propagate(n_pad, fh, nchunk, npass, const_rows=False):
  rows_per_tile = n_pad // _NSUB
  ntab = 2 * npass
  mesh = plsc.VectorSubcoreMesh(core_axis_name="c", subcore_axis_name="s")

  @functools.partial(
      pl.kernel,
      out_type=[jax.ShapeDtypeStruct((n_pad, fh), jnp.float32)] * ntab,
      mesh=mesh,
      scratch_types=[
          pltpu.VMEM((nchunk, _CH), jnp.int32),
          pltpu.VMEM((nchunk, _CH), jnp.int32),
          pltpu.VMEM((5, _CH, fh), jnp.float32),
          pltpu.VMEM((_ZB, fh), jnp.float32),
          pltpu.VMEM_SHARED((n_pad, fh), jnp.float32),
          pltpu.SemaphoreType.DMA,
          pltpu.SemaphoreType.DMA,
      ],
      compiler_params=pltpu.CompilerParams(use_tc_tiling_on_sc=False),
  )
  def prop(*refs):
    tables = refs[:ntab]
    srcb, dstb, zrows = refs[ntab:ntab + 3]
    outs = refs[ntab + 3:2 * ntab + 3]
    src_v, dst_v, rows_v, zero_v, acc_sh, sem_g, sem_s = refs[2 * ntab + 3:]

    c = lax.axis_index("c")
    s = lax.axis_index("s")
    base = s * rows_per_tile
    sl = pl.ds(base, rows_per_tile)

    pltpu.sync_copy(zrows, zero_v)
    pltpu.sync_copy(srcb.at[s], src_v)
    pltpu.sync_copy(dstb.at[s], dst_v)

    def run(tbl):
      def gather(j):
        pltpu.async_copy(tbl.at[src_v.at[j]], rows_v.at[j % 5], sem_g)

      def wait_gather(j):
        pltpu.make_async_copy(tbl.at[src_v.at[j]], rows_v.at[j % 5],
                              sem_g).wait()

      def scatter(j, buf):
        pltpu.async_copy(rows_v.at[buf], acc_sh.at[dst_v.at[j]], sem_s,
                         add=True)

      def wait_scatter(j, buf):
        pltpu.make_async_copy(rows_v.at[buf], acc_sh.at[dst_v.at[j]],
                              sem_s).wait()

      if const_rows:
        pltpu.sync_copy(tbl, rows_v.at[0])

        @pl.loop(0, nchunk)
        def _(j):
          @pl.when(j >= 3)
          def _():
            wait_scatter(j - 3, 0)

          scatter(j, 0)

        @pl.loop(max(nchunk - 3, 0), nchunk)
        def _(j):
          wait_scatter(j, 0)
        return

      for jj in range(min(2, nchunk)):
        gather(jj)

      @pl.loop(0, nchunk)
      def _(j):
        @pl.when(j >= 3)
        def _():
          wait_scatter(j - 3, (j - 3) % 5)

        @pl.when(j + 2 < nchunk)
        def _():
          gather(j + 2)

        wait_gather(j)
        scatter(j, j % 5)

      @pl.loop(max(nchunk - 3, 0), nchunk)
      def _(j):
        wait_scatter(j, j % 5)

    for p in range(npass):
      @pl.loop(0, rows_per_tile // _ZB)
      def _(i):
        pltpu.sync_copy(zero_v, acc_sh.at[pl.ds(base + i * _ZB, _ZB)])

      plsc.subcore_barrier()

      @pl.when(c == 0)
      def _():
        run(tables[p])

      @pl.when(c == 1)
      def _():
        run(tables[npass + p])

      plsc.subcore_barrier()

      @pl.when(c == 0)
      def _():
        pltpu.sync_copy(acc_sh.at[sl], outs[p].at[sl])

      @pl.when(c == 1)
      def _():
        pltpu.sync_copy(acc_sh.at[sl], outs[npass + p].at[sl])

  return prop


def _dot(a, b):
  return jnp.dot(a, b, preferred_element_type=jnp.float32)


def _row_spec(cols):
  return pl.BlockSpec((_BLK, cols), lambda i: (i, 0))


def _full_spec(r, c):
  return pl.BlockSpec((r, c), lambda i: (0, 0))


def _split(u, nout):
  p = u.shape[1] // nout
  return [u[:, k * p:(k + 1) * p] for k in range(nout)]


@functools.lru_cache(maxsize=None)
def _make_tc0(n_pad, f_in, nout):
  nb = n_pad // _BLK
  q = f_in // nout

  def body(x_ref, deg_ref, dinv_ref, *xt_refs):
    cnt = deg_ref[:, 0:1]
    dinv = lax.rsqrt(cnt + 1.0)
    dinv_ref[...] = jnp.broadcast_to(dinv, (_BLK, 128))
    xt = x_ref[...] * dinv
    for r, piece in zip(xt_refs, _split(xt, nout)):
      r[...] = piece

  return pl.pallas_call(
      body,
      grid=(nb,),
      in_specs=[_row_spec(f_in), _row_spec(16)],
      out_specs=[_row_spec(128)] + [_row_spec(q)] * nout,
      out_shape=[jax.ShapeDtypeStruct((n_pad, 128), jnp.float32)]
      + [jax.ShapeDtypeStruct((n_pad, q), jnp.float32)] * nout,
  )


@functools.lru_cache(maxsize=None)
def _make_tc01(n_pad, f_in, h_dim, nin, nout):
  nb = n_pad // _BLK
  qi = f_in // nin
  qo = h_dim // nout

  def body(*refs):
    it = iter(refs)
    accs = [next(it) for _ in range(nin)]
    xts = [next(it) for _ in range(nin)]
    dinv_ref, b_ref, w0_ref, w1_ref = next(it), next(it), next(it), next(it)
    outs = list(it)

    dinv = dinv_ref[:, 0:1]
    p = jnp.concatenate([dinv * (a[...] + t[...]) for a, t in zip(accs, xts)],
                        axis=1)
    h1 = jnp.maximum(_dot(p, w0_ref[...]) + b_ref[...], 0.0)
    for r, piece in zip(outs[:nout], _split(h1, nout)):
      r[...] = piece
    u1 = _dot(h1, w1_ref[...]) * dinv
    for r, piece in zip(outs[nout:], _split(u1, nout)):
      r[...] = piece

  in_specs = [_row_spec(qi)] * (2 * nin)
  in_specs += [_row_spec(128), _full_spec(1, h_dim),
               _full_spec(f_in, h_dim), _full_spec(h_dim, h_dim)]
  out_specs = [_row_spec(qo)] * (2 * nout)
  out_shape = [jax.ShapeDtypeStruct((n_pad, qo), jnp.float32)] * (2 * nout)

  return pl.pallas_call(
      body, grid=(nb,), in_specs=in_specs, out_specs=out_specs,
      out_shape=out_shape)


@functools.lru_cache(maxsize=None)
def _make_tc_layer(n_pad, h_dim, w_cols, nin, nout, with_resid, emit_h):
  nb = n_pad // _BLK
  qi = h_dim // nin
  qo = w_cols // nout

  def body(*refs):
    it = iter(refs)
    accs = [next(it) for _ in range(nin)]
    us = [next(it) for _ in range(nin)]
    rs = [next(it) for _ in range(nin)] if with_resid else None
    dinv_ref, b_ref, w_ref = next(it), next(it), next(it)
    outs = list(it)

    dinv = dinv_ref[:, 0:1]
    b = b_ref[...]
    hs = []
    for k in range(nin):
      hk = jnp.maximum(
          dinv * (accs[k][...] + us[k][...]) + b[:, k * qi:(k + 1) * qi], 0.0)
      if with_resid:
        hk = hk + rs[k][...]
      hs.append(hk)
    if emit_h:
      for r, hk in zip(outs[:nin], hs):
        r[...] = hk
      outs = outs[nin:]
    h = jnp.concatenate(hs, axis=1)
    u = _dot(h, w_ref[...]) * dinv
    for r, piece in zip(outs, _split(u, nout)):
      r[...] = piece

  in_specs = [_row_spec(qi)] * (nin * (3 if with_resid else 2))
  in_specs += [_row_spec(128), _full_spec(1, h_dim), _full_spec(h_dim, w_cols)]

  out_specs = []
  out_shape = []
  if emit_h:
    out_specs += [_row_spec(qi)] * nin
    out_shape += [jax.ShapeDtypeStruct((n_pad, qi), jnp.float32)] * nin
  out_specs += [_row_spec(qo)] * nout
  out_shape += [jax.ShapeDtypeStruct((n_pad, qo), jnp.float32)] * nout

  return pl.pallas_call(
      body, grid=(nb,), in_specs=in_specs, out_specs=out_specs,
      out_shape=out_shape)


@functools.lru_cache(maxsize=None)
def _make_tc_pool(n_pad, c_dim, nin):
  nb = n_pad // _BLK
  qi = c_dim // nin

  def body(*refs):
    it = iter(refs)
    accs = [next(it) for _ in range(nin)]
    us = [next(it) for _ in range(nin)]
    dinv_ref, b_ref, bt_ref, out_ref, pooled = (next(it), next(it), next(it),
                                                next(it), next(it))
    i = pl.program_id(0)

    @pl.when(i == 0)
    def _():
      pooled[...] = jnp.zeros((_G, 128), jnp.float32)

    dinv = dinv_ref[:, 0:1]
    h = dinv * jnp.concatenate([a[...] + u[...] for a, u in zip(accs, us)],
                               axis=1) + b_ref[...]
    xc = jnp.concatenate([h, jnp.ones((_BLK, 128 - c_dim), jnp.float32)],
                         axis=1)
    onehot = (bt_ref[...] == lax.broadcasted_iota(jnp.int32, (_BLK, _G), 1)
              ).astype(jnp.float32)
    pooled[...] += lax.dot_general(
        onehot, xc, (((0,), (0,)), ((), ())),
        preferred_element_type=jnp.float32)

    @pl.when(i == nb - 1)
    def _():
      p = pooled[...]
      mean = p[:, :c_dim] / jnp.maximum(p[:, c_dim:c_dim + 1], 1.0)
      z = mean - jnp.max(mean, axis=1, keepdims=True)
      lse = jnp.log(jnp.sum(jnp.exp(z), axis=1, keepdims=True))
      out_ref[...] = z - lse

  return pl.pallas_call(
      body,
      grid=(nb,),
      in_specs=[_row_spec(qi)] * (2 * nin)
      + [_row_spec(128), _full_spec(1, c_dim),
         pl.BlockSpec((_BLK, 1), lambda i: (i, 0))],
      out_specs=pl.BlockSpec((_G, c_dim), lambda i: (0, 0)),
      out_shape=jax.ShapeDtypeStruct((_G, c_dim), jnp.float32),
      scratch_shapes=[pltpu.VMEM((_G, 128), jnp.float32)],
  )


def kernel(x, edge_index, batch, W0, b0, W1, b1, W2, b2, W3, b3):
  n, f_in = x.shape
  e = edge_index.shape[1]
  h_dim = W0.shape[1]
  c_dim = W3.shape[1]

  n_pad = -(-n // 2048) * 2048
  nchunk = -(-e // (_NSUB * _CH))
  ep = _NSUB * nchunk * _CH

  pad_idx = jnp.full((ep - e,), n, jnp.int32)
  srcb = jnp.concatenate([edge_index[0].astype(jnp.int32), pad_idx]
                         ).reshape(_NSUB, nchunk, _CH)
  dstb = jnp.concatenate([edge_index[1].astype(jnp.int32), pad_idx]
                         ).reshape(_NSUB, nchunk, _CH)

  xp = jnp.pad(x, ((0, n_pad - n), (0, 0)))
  bt = jnp.pad(batch.astype(jnp.int32), (0, n_pad - n),
               constant_values=_G).reshape(n_pad, 1)

  ones16 = jnp.ones((_CH, 16), jnp.float32)
  z16 = jnp.zeros((_ZB, 16), jnp.float32)
  zh = jnp.zeros((_ZB, h_dim // 4), jnp.float32)
  zc = jnp.zeros((_ZB, c_dim // 2), jnp.float32)

  prop_deg = _make_propagate(n_pad, 16, nchunk, 1, const_rows=True)
  prop_x = _make_propagate(n_pad, f_in // 2, nchunk, 1)
  prop_h = _make_propagate(n_pad, h_dim // 4, nchunk, 2)
  prop_c = _make_propagate(n_pad, c_dim // 2, nchunk, 1)

  tc0 = _make_tc0(n_pad, f_in, 2)
  tc1 = _make_tc01(n_pad, f_in, h_dim, 2, 4)
  tc2 = _make_tc_layer(n_pad, h_dim, h_dim, 4, 4, True, False)
  tc3 = _make_tc_layer(n_pad, h_dim, c_dim, 4, 2, True, False)
  tc4 = _make_tc_pool(n_pad, c_dim, 2)

  b0r = b0.reshape(1, -1)
  b1r = b1.reshape(1, -1)
  b2r = b2.reshape(1, -1)
  b3r = b3.reshape(1, -1)

  zx = jnp.zeros((_ZB, f_in // 2), jnp.float32)
  deg = prop_deg(ones16, ones16, srcb, dstb, z16)[0]
  dinv_b, *xt = tc0(xp, deg)

  ax = prop_x(*xt, srcb, dstb, zx)
  h1a, h1b, h1c, h1d, *u1 = tc1(*ax, *xt, dinv_b, b0r, W0, W1)

  a1 = prop_h(*u1, srcb, dstb, zh)
  u2 = tc2(*a1, *u1, h1a, h1b, h1c, h1d, dinv_b, b1r, W2)

  a2 = prop_h(*u2, srcb, dstb, zh)
  u3 = tc3(*a2, *u2, h1a, h1b, h1c, h1d, dinv_b, b2r, W3)

  a3 = prop_c(*u3, srcb, dstb, zc)
  return tc4(*a3, *u3, dinv_b, b3r, bt)


# speedup vs baseline: 2.2327x; 1.0129x over previous
import functools

import jax
import jax.numpy as jnp
from jax import lax
from jax.experimental import pallas as pl
from jax.experimental.pallas import tpu as pltpu
from jax.experimental.pallas import tpu_sc as plsc

_CH = 128
_ZB = 64
_NSUB = 16
_BLK = 512
_G = 64


@functools.lru_cache(maxsize=None)
def _make_propagate(n_pad, fh, nchunk, npass, const_rows=False):
  rows_per_tile = n_pad // _NSUB
  ntab = 2 * npass
  mesh = plsc.VectorSubcoreMesh(core_axis_name="c", subcore_axis_name="s")

  @functools.partial(
      pl.kernel,
      out_type=[jax.ShapeDtypeStruct((n_pad, fh), jnp.float32)] * ntab,
      mesh=mesh,
      scratch_types=[
          pltpu.VMEM((nchunk, _CH), jnp.int32),
          pltpu.VMEM((nchunk, _CH), jnp.int32),
          pltpu.VMEM((5, _CH, fh), jnp.float32),
          pltpu.VMEM((_ZB, fh), jnp.float32),
          pltpu.VMEM_SHARED((n_pad, fh), jnp.float32),
          pltpu.SemaphoreType.DMA,
          pltpu.SemaphoreType.DMA,
      ],
      compiler_params=pltpu.CompilerParams(use_tc_tiling_on_sc=False),
  )
  def prop(*refs):
    tables = refs[:ntab]
    srcb, dstb, zrows = refs[ntab:ntab + 3]
    outs = refs[ntab + 3:2 * ntab + 3]
    src_v, dst_v, rows_v, zero_v, acc_sh, sem_g, sem_s = refs[2 * ntab + 3:]

    c = lax.axis_index("c")
    s = lax.axis_index("s")
    base = s * rows_per_tile
    sl = pl.ds(base, rows_per_tile)

    pltpu.async_copy(srcb.at[s], src_v, sem_g)
    pltpu.async_copy(dstb.at[s], dst_v, sem_g)
    pltpu.sync_copy(zrows, zero_v)

    def zero_acc():
      @pl.loop(0, rows_per_tile // _ZB)
      def _(i):
        pltpu.async_copy(zero_v, acc_sh.at[pl.ds(base + i * _ZB, _ZB)],
                         sem_s)

      @pl.loop(0, rows_per_tile // _ZB)
      def _(i):
        pltpu.make_async_copy(zero_v, acc_sh.at[pl.ds(base + i * _ZB, _ZB)],
                              sem_s).wait()

    zero_acc()
    pltpu.make_async_copy(srcb.at[s], src_v, sem_g).wait()
    pltpu.make_async_copy(dstb.at[s], dst_v, sem_g).wait()

    def run(tbl):
      def gather(j):
        pltpu.async_copy(tbl.at[src_v.at[j]], rows_v.at[j % 5], sem_g)

      def wait_gather(j):
        pltpu.make_async_copy(tbl.at[src_v.at[j]], rows_v.at[j % 5],
                              sem_g).wait()

      def scatter(j, buf):
        pltpu.async_copy(rows_v.at[buf], acc_sh.at[dst_v.at[j]], sem_s,
                         add=True)

      def wait_scatter(j, buf):
        pltpu.make_async_copy(rows_v.at[buf], acc_sh.at[dst_v.at[j]],
                              sem_s).wait()

      if const_rows:
        pltpu.sync_copy(tbl, rows_v.at[0])

        @pl.loop(0, nchunk)
        def _(j):
          @pl.when(j >= 3)
          def _():
            wait_scatter(j - 3, 0)

          scatter(j, 0)

        @pl.loop(max(nchunk - 3, 0), nchunk)
        def _(j):
          wait_scatter(j, 0)
        return

      for jj in range(min(2, nchunk)):
        gather(jj)

      @pl.loop(0, nchunk)
      def _(j):
        @pl.when(j >= 3)
        def _():
          wait_scatter(j - 3, (j - 3) % 5)

        @pl.when(j + 2 < nchunk)
        def _():
          gather(j + 2)

        wait_gather(j)
        scatter(j, j % 5)

      @pl.loop(max(nchunk - 3, 0), nchunk)
      def _(j):
        wait_scatter(j, j % 5)

    for p in range(npass):
      if p:
        zero_acc()

      plsc.subcore_barrier()

      @pl.when(c == 0)
      def _():
        run(tables[p])

      @pl.when(c == 1)
      def _():
        run(tables[npass + p])

      plsc.subcore_barrier()

      @pl.when(c == 0)
      def _():
        pltpu.sync_copy(acc_sh.at[sl], outs[p].at[sl])

      @pl.when(c == 1)
      def _():
        pltpu.sync_copy(acc_sh.at[sl], outs[npass + p].at[sl])

  return prop


def _dot(a, b):
  return jnp.dot(a, b, preferred_element_type=jnp.float32)


def _row_spec(cols):
  return pl.BlockSpec((_BLK, cols), lambda i: (i, 0))


def _full_spec(r, c):
  return pl.BlockSpec((r, c), lambda i: (0, 0))


def _split(u, nout):
  p = u.shape[1] // nout
  return [u[:, k * p:(k + 1) * p] for k in range(nout)]


@functools.lru_cache(maxsize=None)
def _make_tc0(n_pad, f_in, nout):
  nb = n_pad // _BLK
  q = f_in // nout

  def body(x_ref, deg_ref, dinv_ref, *xt_refs):
    cnt = deg_ref[:, 0:1]
    dinv = lax.rsqrt(cnt + 1.0)
    dinv_ref[...] = jnp.broadcast_to(dinv, (_BLK, 128))
    xt = x_ref[...] * dinv
    for r, piece in zip(xt_refs, _split(xt, nout)):
      r[...] = piece

  return pl.pallas_call(
      body,
      grid=(nb,),
      in_specs=[_row_spec(f_in), _row_spec(16)],
      out_specs=[_row_spec(128)] + [_row_spec(q)] * nout,
      out_shape=[jax.ShapeDtypeStruct((n_pad, 128), jnp.float32)]
      + [jax.ShapeDtypeStruct((n_pad, q), jnp.float32)] * nout,
  )


@functools.lru_cache(maxsize=None)
def _make_tc01(n_pad, f_in, h_dim, nin, nout):
  nb = n_pad // _BLK
  qi = f_in // nin
  qo = h_dim // nout

  def body(*refs):
    it = iter(refs)
    accs = [next(it) for _ in range(nin)]
    xts = [next(it) for _ in range(nin)]
    dinv_ref, b_ref, w0_ref, w1_ref = next(it), next(it), next(it), next(it)
    outs = list(it)

    dinv = dinv_ref[:, 0:1]
    p = jnp.concatenate([dinv * (a[...] + t[...]) for a, t in zip(accs, xts)],
                        axis=1)
    h1 = jnp.maximum(_dot(p, w0_ref[...]) + b_ref[...], 0.0)
    for r, piece in zip(outs[:nout], _split(h1, nout)):
      r[...] = piece
    u1 = _dot(h1, w1_ref[...]) * dinv
    for r, piece in zip(outs[nout:], _split(u1, nout)):
      r[...] = piece

  in_specs = [_row_spec(qi)] * (2 * nin)
  in_specs += [_row_spec(128), _full_spec(1, h_dim),
               _full_spec(f_in, h_dim), _full_spec(h_dim, h_dim)]
  out_specs = [_row_spec(qo)] * (2 * nout)
  out_shape = [jax.ShapeDtypeStruct((n_pad, qo), jnp.float32)] * (2 * nout)

  return pl.pallas_call(
      body, grid=(nb,), in_specs=in_specs, out_specs=out_specs,
      out_shape=out_shape)


@functools.lru_cache(maxsize=None)
def _make_tc_layer(n_pad, h_dim, w_cols, nin, nout, with_resid, emit_h):
  nb = n_pad // _BLK
  qi = h_dim // nin
  qo = w_cols // nout

  def body(*refs):
    it = iter(refs)
    accs = [next(it) for _ in range(nin)]
    us = [next(it) for _ in range(nin)]
    rs = [next(it) for _ in range(nin)] if with_resid else None
    dinv_ref, b_ref, w_ref = next(it), next(it), next(it)
    outs = list(it)

    dinv = dinv_ref[:, 0:1]
    b = b_ref[...]
    hs = []
    for k in range(nin):
      hk = jnp.maximum(
          dinv * (accs[k][...] + us[k][...]) + b[:, k * qi:(k + 1) * qi], 0.0)
      if with_resid:
        hk = hk + rs[k][...]
      hs.append(hk)
    if emit_h:
      for r, hk in zip(outs[:nin], hs):
        r[...] = hk
      outs = outs[nin:]
    h = jnp.concatenate(hs, axis=1)
    u = _dot(h, w_ref[...]) * dinv
    for r, piece in zip(outs, _split(u, nout)):
      r[...] = piece

  in_specs = [_row_spec(qi)] * (nin * (3 if with_resid else 2))
  in_specs += [_row_spec(128), _full_spec(1, h_dim), _full_spec(h_dim, w_cols)]

  out_specs = []
  out_shape = []
  if emit_h:
    out_specs += [_row_spec(qi)] * nin
    out_shape += [jax.ShapeDtypeStruct((n_pad, qi), jnp.float32)] * nin
  out_specs += [_row_spec(qo)] * nout
  out_shape += [jax.ShapeDtypeStruct((n_pad, qo), jnp.float32)] * nout

  return pl.pallas_call(
      body, grid=(nb,), in_specs=in_specs, out_specs=out_specs,
      out_shape=out_shape)


@functools.lru_cache(maxsize=None)
def _make_tc_pool(n_pad, c_dim, nin):
  nb = n_pad // _BLK
  qi = c_dim // nin

  def body(*refs):
    it = iter(refs)
    accs = [next(it) for _ in range(nin)]
    us = [next(it) for _ in range(nin)]
    dinv_ref, b_ref, bt_ref, out_ref, pooled = (next(it), next(it), next(it),
                                                next(it), next(it))
    i = pl.program_id(0)

    @pl.when(i == 0)
    def _():
      pooled[...] = jnp.zeros((_G, 128), jnp.float32)

    dinv = dinv_ref[:, 0:1]
    h = dinv * jnp.concatenate([a[...] + u[...] for a, u in zip(accs, us)],
                               axis=1) + b_ref[...]
    xc = jnp.concatenate([h, jnp.ones((_BLK, 128 - c_dim), jnp.float32)],
                         axis=1)
    onehot = (bt_ref[...] == lax.broadcasted_iota(jnp.int32, (_BLK, _G), 1)
              ).astype(jnp.float32)
    pooled[...] += lax.dot_general(
        onehot, xc, (((0,), (0,)), ((), ())),
        preferred_element_type=jnp.float32)

    @pl.when(i == nb - 1)
    def _():
      p = pooled[...]
      mean = p[:, :c_dim] / jnp.maximum(p[:, c_dim:c_dim + 1], 1.0)
      z = mean - jnp.max(mean, axis=1, keepdims=True)
      lse = jnp.log(jnp.sum(jnp.exp(z), axis=1, keepdims=True))
      out_ref[...] = z - lse

  return pl.pallas_call(
      body,
      grid=(nb,),
      in_specs=[_row_spec(qi)] * (2 * nin)
      + [_row_spec(128), _full_spec(1, c_dim),
         pl.BlockSpec((_BLK, 1), lambda i: (i, 0))],
      out_specs=pl.BlockSpec((_G, c_dim), lambda i: (0, 0)),
      out_shape=jax.ShapeDtypeStruct((_G, c_dim), jnp.float32),
      scratch_shapes=[pltpu.VMEM((_G, 128), jnp.float32)],
  )


def kernel(x, edge_index, batch, W0, b0, W1, b1, W2, b2, W3, b3):
  n, f_in = x.shape
  e = edge_index.shape[1]
  h_dim = W0.shape[1]
  c_dim = W3.shape[1]

  n_pad = -(-n // 2048) * 2048
  nchunk = -(-e // (_NSUB * _CH))
  ep = _NSUB * nchunk * _CH

  pad_idx = jnp.full((ep - e,), n, jnp.int32)
  srcb = jnp.concatenate([edge_index[0].astype(jnp.int32), pad_idx]
                         ).reshape(_NSUB, nchunk, _CH)
  dstb = jnp.concatenate([edge_index[1].astype(jnp.int32), pad_idx]
                         ).reshape(_NSUB, nchunk, _CH)

  xp = jnp.pad(x, ((0, n_pad - n), (0, 0)))
  bt = jnp.pad(batch.astype(jnp.int32), (0, n_pad - n),
               constant_values=_G).reshape(n_pad, 1)

  ones16 = jnp.ones((_CH, 16), jnp.float32)
  z16 = jnp.zeros((_ZB, 16), jnp.float32)
  zh = jnp.zeros((_ZB, h_dim // 4), jnp.float32)
  zc = jnp.zeros((_ZB, c_dim // 2), jnp.float32)

  prop_deg = _make_propagate(n_pad, 16, nchunk, 1, const_rows=True)
  prop_x = _make_propagate(n_pad, f_in // 2, nchunk, 1)
  prop_h = _make_propagate(n_pad, h_dim // 4, nchunk, 2)
  prop_c = _make_propagate(n_pad, c_dim // 2, nchunk, 1)

  tc0 = _make_tc0(n_pad, f_in, 2)
  tc1 = _make_tc01(n_pad, f_in, h_dim, 2, 4)
  tc2 = _make_tc_layer(n_pad, h_dim, h_dim, 4, 4, True, False)
  tc3 = _make_tc_layer(n_pad, h_dim, c_dim, 4, 2, True, False)
  tc4 = _make_tc_pool(n_pad, c_dim, 2)

  b0r = b0.reshape(1, -1)
  b1r = b1.reshape(1, -1)
  b2r = b2.reshape(1, -1)
  b3r = b3.reshape(1, -1)

  zx = jnp.zeros((_ZB, f_in // 2), jnp.float32)
  deg = prop_deg(ones16, ones16, srcb, dstb, z16)[0]
  dinv_b, *xt = tc0(xp, deg)

  ax = prop_x(*xt, srcb, dstb, zx)
  h1a, h1b, h1c, h1d, *u1 = tc1(*ax, *xt, dinv_b, b0r, W0, W1)

  a1 = prop_h(*u1, srcb, dstb, zh)
  u2 = tc2(*a1, *u1, h1a, h1b, h1c, h1d, dinv_b, b1r, W2)

  a2 = prop_h(*u2, srcb, dstb, zh)
  u3 = tc3(*a2, *u2, h1a, h1b, h1c, h1d, dinv_b, b2r, W3)

  a3 = prop_c(*u3, srcb, dstb, zc)
  return tc4(*a3, *u3, dinv_b, b3r, bt)


# speedup vs baseline: 2.2504x; 1.0080x over previous
import functools

import jax
import jax.numpy as jnp
from jax import lax
from jax.experimental import pallas as pl
from jax.experimental.pallas import tpu as pltpu
from jax.experimental.pallas import tpu_sc as plsc

_CH = 128
_ZB = 64
_NSUB = 16
_BLK = 512
_G = 64


@functools.lru_cache(maxsize=None)
def _make_propagate(n_pad, fh, nchunk, npass, const_rows=False):
  rows_per_tile = n_pad // _NSUB
  ntab = 2 * npass
  mesh = plsc.VectorSubcoreMesh(core_axis_name="c", subcore_axis_name="s")

  @functools.partial(
      pl.kernel,
      out_type=[jax.ShapeDtypeStruct((n_pad, fh), jnp.float32)] * ntab,
      mesh=mesh,
      scratch_types=[
          pltpu.VMEM((nchunk, _CH), jnp.int32),
          pltpu.VMEM((nchunk, _CH), jnp.int32),
          pltpu.VMEM((5, _CH, fh), jnp.float32),
          pltpu.VMEM((_ZB, fh), jnp.float32),
          pltpu.VMEM_SHARED((n_pad, fh), jnp.float32),
          pltpu.SemaphoreType.DMA,
          pltpu.SemaphoreType.DMA,
      ],
      compiler_params=pltpu.CompilerParams(use_tc_tiling_on_sc=False),
  )
  def prop(*refs):
    tables = refs[:ntab]
    srcb, dstb, zrows = refs[ntab:ntab + 3]
    outs = refs[ntab + 3:2 * ntab + 3]
    src_v, dst_v, rows_v, zero_v, acc_sh, sem_g, sem_s = refs[2 * ntab + 3:]

    c = lax.axis_index("c")
    s = lax.axis_index("s")
    base = s * rows_per_tile
    sl = pl.ds(base, rows_per_tile)

    pltpu.async_copy(srcb.at[s], src_v, sem_g)
    pltpu.async_copy(dstb.at[s], dst_v, sem_g)
    pltpu.sync_copy(zrows, zero_v)

    def zero_acc():
      @pl.loop(0, rows_per_tile // _ZB)
      def _(i):
        pltpu.async_copy(zero_v, acc_sh.at[pl.ds(base + i * _ZB, _ZB)],
                         sem_s)

      @pl.loop(0, rows_per_tile // _ZB)
      def _(i):
        pltpu.make_async_copy(zero_v, acc_sh.at[pl.ds(base + i * _ZB, _ZB)],
                              sem_s).wait()

    zero_acc()
    pltpu.make_async_copy(srcb.at[s], src_v, sem_g).wait()
    pltpu.make_async_copy(dstb.at[s], dst_v, sem_g).wait()

    def run(tbl):
      def gather(j):
        pltpu.async_copy(tbl.at[src_v.at[j]], rows_v.at[j % 5], sem_g)

      def wait_gather(j):
        pltpu.make_async_copy(tbl.at[src_v.at[j]], rows_v.at[j % 5],
                              sem_g).wait()

      def scatter(j, buf):
        pltpu.async_copy(rows_v.at[buf], acc_sh.at[dst_v.at[j]], sem_s,
                         add=True)

      def wait_scatter(j, buf):
        pltpu.make_async_copy(rows_v.at[buf], acc_sh.at[dst_v.at[j]],
                              sem_s).wait()

      if const_rows:
        pltpu.sync_copy(tbl, rows_v.at[0])

        @pl.loop(0, nchunk)
        def _(j):
          @pl.when(j >= 3)
          def _():
            wait_scatter(j - 3, 0)

          scatter(j, 0)

        @pl.loop(max(nchunk - 3, 0), nchunk)
        def _(j):
          wait_scatter(j, 0)
        return

      for jj in range(min(3, nchunk)):
        gather(jj)

      @pl.loop(0, nchunk)
      def _(j):
        @pl.when(j >= 2)
        def _():
          wait_scatter(j - 2, (j - 2) % 5)

        @pl.when(j + 3 < nchunk)
        def _():
          gather(j + 3)

        wait_gather(j)
        scatter(j, j % 5)

      @pl.loop(max(nchunk - 2, 0), nchunk)
      def _(j):
        wait_scatter(j, j % 5)

    for p in range(npass):
      if p:
        zero_acc()

      plsc.subcore_barrier()

      @pl.when(c == 0)
      def _():
        run(tables[p])

      @pl.when(c == 1)
      def _():
        run(tables[npass + p])

      plsc.subcore_barrier()

      @pl.when(c == 0)
      def _():
        pltpu.sync_copy(acc_sh.at[sl], outs[p].at[sl])

      @pl.when(c == 1)
      def _():
        pltpu.sync_copy(acc_sh.at[sl], outs[npass + p].at[sl])

  return prop


def _dot(a, b):
  return jnp.dot(a, b, preferred_element_type=jnp.float32)


def _row_spec(cols):
  return pl.BlockSpec((_BLK, cols), lambda i: (i, 0))


def _full_spec(r, c):
  return pl.BlockSpec((r, c), lambda i: (0, 0))


def _split(u, nout):
  p = u.shape[1] // nout
  return [u[:, k * p:(k + 1) * p] for k in range(nout)]


@functools.lru_cache(maxsize=None)
def _make_tc0(n_pad, f_in, nout):
  nb = n_pad // _BLK
  q = f_in // nout

  def body(x_ref, deg_ref, dinv_ref, *xt_refs):
    cnt = deg_ref[:, 0:1]
    dinv = lax.rsqrt(cnt + 1.0)
    dinv_ref[...] = jnp.broadcast_to(dinv, (_BLK, 128))
    xt = x_ref[...] * dinv
    for r, piece in zip(xt_refs, _split(xt, nout)):
      r[...] = piece

  return pl.pallas_call(
      body,
      grid=(nb,),
      in_specs=[_row_spec(f_in), _row_spec(16)],
      out_specs=[_row_spec(128)] + [_row_spec(q)] * nout,
      out_shape=[jax.ShapeDtypeStruct((n_pad, 128), jnp.float32)]
      + [jax.ShapeDtypeStruct((n_pad, q), jnp.float32)] * nout,
  )


@functools.lru_cache(maxsize=None)
def _make_tc01(n_pad, f_in, h_dim, nin, nout):
  nb = n_pad // _BLK
  qi = f_in // nin
  qo = h_dim // nout

  def body(*refs):
    it = iter(refs)
    accs = [next(it) for _ in range(nin)]
    xts = [next(it) for _ in range(nin)]
    dinv_ref, b_ref, w0_ref, w1_ref = next(it), next(it), next(it), next(it)
    outs = list(it)

    dinv = dinv_ref[:, 0:1]
    p = jnp.concatenate([dinv * (a[...] + t[...]) for a, t in zip(accs, xts)],
                        axis=1)
    h1 = jnp.maximum(_dot(p, w0_ref[...]) + b_ref[...], 0.0)
    for r, piece in zip(outs[:nout], _split(h1, nout)):
      r[...] = piece
    u1 = _dot(h1, w1_ref[...]) * dinv
    for r, piece in zip(outs[nout:], _split(u1, nout)):
      r[...] = piece

  in_specs = [_row_spec(qi)] * (2 * nin)
  in_specs += [_row_spec(128), _full_spec(1, h_dim),
               _full_spec(f_in, h_dim), _full_spec(h_dim, h_dim)]
  out_specs = [_row_spec(qo)] * (2 * nout)
  out_shape = [jax.ShapeDtypeStruct((n_pad, qo), jnp.float32)] * (2 * nout)

  return pl.pallas_call(
      body, grid=(nb,), in_specs=in_specs, out_specs=out_specs,
      out_shape=out_shape)


@functools.lru_cache(maxsize=None)
def _make_tc_layer(n_pad, h_dim, w_cols, nin, nout, with_resid, emit_h):
  nb = n_pad // _BLK
  qi = h_dim // nin
  qo = w_cols // nout

  def body(*refs):
    it = iter(refs)
    accs = [next(it) for _ in range(nin)]
    us = [next(it) for _ in range(nin)]
    rs = [next(it) for _ in range(nin)] if with_resid else None
    dinv_ref, b_ref, w_ref = next(it), next(it), next(it)
    outs = list(it)

    dinv = dinv_ref[:, 0:1]
    b = b_ref[...]
    hs = []
    for k in range(nin):
      hk = jnp.maximum(
          dinv * (accs[k][...] + us[k][...]) + b[:, k * qi:(k + 1) * qi], 0.0)
      if with_resid:
        hk = hk + rs[k][...]
      hs.append(hk)
    if emit_h:
      for r, hk in zip(outs[:nin], hs):
        r[...] = hk
      outs = outs[nin:]
    h = jnp.concatenate(hs, axis=1)
    u = _dot(h, w_ref[...]) * dinv
    for r, piece in zip(outs, _split(u, nout)):
      r[...] = piece

  in_specs = [_row_spec(qi)] * (nin * (3 if with_resid else 2))
  in_specs += [_row_spec(128), _full_spec(1, h_dim), _full_spec(h_dim, w_cols)]

  out_specs = []
  out_shape = []
  if emit_h:
    out_specs += [_row_spec(qi)] * nin
    out_shape += [jax.ShapeDtypeStruct((n_pad, qi), jnp.float32)] * nin
  out_specs += [_row_spec(qo)] * nout
  out_shape += [jax.ShapeDtypeStruct((n_pad, qo), jnp.float32)] * nout

  return pl.pallas_call(
      body, grid=(nb,), in_specs=in_specs, out_specs=out_specs,
      out_shape=out_shape)


@functools.lru_cache(maxsize=None)
def _make_tc_pool(n_pad, c_dim, nin):
  nb = n_pad // _BLK
  qi = c_dim // nin

  def body(*refs):
    it = iter(refs)
    accs = [next(it) for _ in range(nin)]
    us = [next(it) for _ in range(nin)]
    dinv_ref, b_ref, bt_ref, out_ref, pooled = (next(it), next(it), next(it),
                                                next(it), next(it))
    i = pl.program_id(0)

    @pl.when(i == 0)
    def _():
      pooled[...] = jnp.zeros((_G, 128), jnp.float32)

    dinv = dinv_ref[:, 0:1]
    h = dinv * jnp.concatenate([a[...] + u[...] for a, u in zip(accs, us)],
                               axis=1) + b_ref[...]
    xc = jnp.concatenate([h, jnp.ones((_BLK, 128 - c_dim), jnp.float32)],
                         axis=1)
    onehot = (bt_ref[...] == lax.broadcasted_iota(jnp.int32, (_BLK, _G), 1)
              ).astype(jnp.float32)
    pooled[...] += lax.dot_general(
        onehot, xc, (((0,), (0,)), ((), ())),
        preferred_element_type=jnp.float32)

    @pl.when(i == nb - 1)
    def _():
      p = pooled[...]
      mean = p[:, :c_dim] / jnp.maximum(p[:, c_dim:c_dim + 1], 1.0)
      z = mean - jnp.max(mean, axis=1, keepdims=True)
      lse = jnp.log(jnp.sum(jnp.exp(z), axis=1, keepdims=True))
      out_ref[...] = z - lse

  return pl.pallas_call(
      body,
      grid=(nb,),
      in_specs=[_row_spec(qi)] * (2 * nin)
      + [_row_spec(128), _full_spec(1, c_dim),
         pl.BlockSpec((_BLK, 1), lambda i: (i, 0))],
      out_specs=pl.BlockSpec((_G, c_dim), lambda i: (0, 0)),
      out_shape=jax.ShapeDtypeStruct((_G, c_dim), jnp.float32),
      scratch_shapes=[pltpu.VMEM((_G, 128), jnp.float32)],
  )


def kernel(x, edge_index, batch, W0, b0, W1, b1, W2, b2, W3, b3):
  n, f_in = x.shape
  e = edge_index.shape[1]
  h_dim = W0.shape[1]
  c_dim = W3.shape[1]

  n_pad = -(-n // 2048) * 2048
  nchunk = -(-e // (_NSUB * _CH))
  ep = _NSUB * nchunk * _CH

  pad_idx = jnp.full((ep - e,), n, jnp.int32)
  srcb = jnp.concatenate([edge_index[0].astype(jnp.int32), pad_idx]
                         ).reshape(_NSUB, nchunk, _CH)
  dstb = jnp.concatenate([edge_index[1].astype(jnp.int32), pad_idx]
                         ).reshape(_NSUB, nchunk, _CH)

  xp = jnp.pad(x, ((0, n_pad - n), (0, 0)))
  bt = jnp.pad(batch.astype(jnp.int32), (0, n_pad - n),
               constant_values=_G).reshape(n_pad, 1)

  ones16 = jnp.ones((_CH, 16), jnp.float32)
  z16 = jnp.zeros((_ZB, 16), jnp.float32)
  zh = jnp.zeros((_ZB, h_dim // 4), jnp.float32)
  zc = jnp.zeros((_ZB, c_dim // 2), jnp.float32)

  prop_deg = _make_propagate(n_pad, 16, nchunk, 1, const_rows=True)
  prop_x = _make_propagate(n_pad, f_in // 2, nchunk, 1)
  prop_h = _make_propagate(n_pad, h_dim // 4, nchunk, 2)
  prop_c = _make_propagate(n_pad, c_dim // 2, nchunk, 1)

  tc0 = _make_tc0(n_pad, f_in, 2)
  tc1 = _make_tc01(n_pad, f_in, h_dim, 2, 4)
  tc2 = _make_tc_layer(n_pad, h_dim, h_dim, 4, 4, True, False)
  tc3 = _make_tc_layer(n_pad, h_dim, c_dim, 4, 2, True, False)
  tc4 = _make_tc_pool(n_pad, c_dim, 2)

  b0r = b0.reshape(1, -1)
  b1r = b1.reshape(1, -1)
  b2r = b2.reshape(1, -1)
  b3r = b3.reshape(1, -1)

  zx = jnp.zeros((_ZB, f_in // 2), jnp.float32)
  deg = prop_deg(ones16, ones16, srcb, dstb, z16)[0]
  dinv_b, *xt = tc0(xp, deg)

  ax = prop_x(*xt, srcb, dstb, zx)
  h1a, h1b, h1c, h1d, *u1 = tc1(*ax, *xt, dinv_b, b0r, W0, W1)

  a1 = prop_h(*u1, srcb, dstb, zh)
  u2 = tc2(*a1, *u1, h1a, h1b, h1c, h1d, dinv_b, b1r, W2)

  a2 = prop_h(*u2, srcb, dstb, zh)
  u3 = tc3(*a2, *u2, h1a, h1b, h1c, h1d, dinv_b, b2r, W3)

  a3 = prop_c(*u3, srcb, dstb, zc)
  return tc4(*a3, *u3, dinv_b, b3r, bt)


# speedup vs baseline: 2.2564x; 1.0027x over previous
import functools

import jax
import jax.numpy as jnp
from jax import lax
from jax.experimental import pallas as pl
from jax.experimental.pallas import tpu as pltpu
from jax.experimental.pallas import tpu_sc as plsc

_CH = 128
_ZB = 64
_NSUB = 16
_BLK = 512
_G = 64


@functools.lru_cache(maxsize=None)
def _make_propagate(n_pad, fh, nchunk, npass, const_rows=False):
  rows_per_tile = n_pad // _NSUB
  ntab = 2 * npass
  mesh = plsc.VectorSubcoreMesh(core_axis_name="c", subcore_axis_name="s")

  @functools.partial(
      pl.kernel,
      out_type=[jax.ShapeDtypeStruct((n_pad, fh), jnp.float32)] * ntab,
      mesh=mesh,
      scratch_types=[
          pltpu.VMEM((nchunk, _CH), jnp.int32),
          pltpu.VMEM((nchunk, _CH), jnp.int32),
          pltpu.VMEM((5, _CH, fh), jnp.float32),
          pltpu.VMEM((_ZB, fh), jnp.float32),
          pltpu.VMEM_SHARED((n_pad, fh), jnp.float32),
          pltpu.SemaphoreType.DMA,
          pltpu.SemaphoreType.DMA,
      ],
      compiler_params=pltpu.CompilerParams(use_tc_tiling_on_sc=False),
  )
  def prop(*refs):
    tables = refs[:ntab]
    srcb, dstb, zrows = refs[ntab:ntab + 3]
    outs = refs[ntab + 3:2 * ntab + 3]
    src_v, dst_v, rows_v, zero_v, acc_sh, sem_g, sem_s = refs[2 * ntab + 3:]

    c = lax.axis_index("c")
    s = lax.axis_index("s")
    base = s * rows_per_tile
    sl = pl.ds(base, rows_per_tile)

    pltpu.async_copy(srcb.at[s], src_v, sem_g)
    pltpu.async_copy(dstb.at[s], dst_v, sem_g)
    pltpu.sync_copy(zrows, zero_v)

    def zero_acc():
      @pl.loop(0, rows_per_tile // _ZB)
      def _(i):
        pltpu.async_copy(zero_v, acc_sh.at[pl.ds(base + i * _ZB, _ZB)],
                         sem_s)

      @pl.loop(0, rows_per_tile // _ZB)
      def _(i):
        pltpu.make_async_copy(zero_v, acc_sh.at[pl.ds(base + i * _ZB, _ZB)],
                              sem_s).wait()

    zero_acc()
    pltpu.make_async_copy(srcb.at[s], src_v, sem_g).wait()
    pltpu.make_async_copy(dstb.at[s], dst_v, sem_g).wait()

    def run(tbl):
      def gather(j):
        pltpu.async_copy(tbl.at[src_v.at[j]], rows_v.at[j % 5], sem_g)

      def wait_gather(j):
        pltpu.make_async_copy(tbl.at[src_v.at[j]], rows_v.at[j % 5],
                              sem_g).wait()

      def scatter(j, buf):
        pltpu.async_copy(rows_v.at[buf], acc_sh.at[dst_v.at[j]], sem_s,
                         add=True)

      def wait_scatter(j, buf):
        pltpu.make_async_copy(rows_v.at[buf], acc_sh.at[dst_v.at[j]],
                              sem_s).wait()

      if const_rows:
        pltpu.sync_copy(tbl, rows_v.at[0])
        half = nchunk // 2
        lo = c * half
        hi = jnp.where(c == 0, half, nchunk)

        @pl.loop(0, nchunk)
        def _(j):
          @pl.when((j >= lo + 3) & (j < hi))
          def _():
            wait_scatter(j - 3, 0)

          @pl.when((j >= lo) & (j < hi))
          def _():
            scatter(j, 0)

        @pl.loop(0, nchunk)
        def _(j):
          @pl.when((j >= hi - 3) & (j >= lo) & (j < hi))
          def _():
            wait_scatter(j, 0)
        return

      for jj in range(min(3, nchunk)):
        gather(jj)

      @pl.loop(0, nchunk)
      def _(j):
        @pl.when(j >= 2)
        def _():
          wait_scatter(j - 2, (j - 2) % 5)

        @pl.when(j + 3 < nchunk)
        def _():
          gather(j + 3)

        wait_gather(j)
        scatter(j, j % 5)

      @pl.loop(max(nchunk - 2, 0), nchunk)
      def _(j):
        wait_scatter(j, j % 5)

    for p in range(npass):
      if p:
        zero_acc()

      plsc.subcore_barrier()

      @pl.when(c == 0)
      def _():
        run(tables[p])

      @pl.when(c == 1)
      def _():
        run(tables[npass + p])

      plsc.subcore_barrier()

      @pl.when(c == 0)
      def _():
        pltpu.sync_copy(acc_sh.at[sl], outs[p].at[sl])

      @pl.when(c == 1)
      def _():
        pltpu.sync_copy(acc_sh.at[sl], outs[npass + p].at[sl])

  return prop


def _dot(a, b):
  return jnp.dot(a, b, preferred_element_type=jnp.float32)


def _row_spec(cols):
  return pl.BlockSpec((_BLK, cols), lambda i: (i, 0))


def _full_spec(r, c):
  return pl.BlockSpec((r, c), lambda i: (0, 0))


def _split(u, nout):
  p = u.shape[1] // nout
  return [u[:, k * p:(k + 1) * p] for k in range(nout)]


@functools.lru_cache(maxsize=None)
def _make_tc0(n_pad, f_in, nout):
  nb = n_pad // _BLK
  q = f_in // nout

  def body(x_ref, deg_ref, deg2_ref, dinv_ref, *xt_refs):
    cnt = deg_ref[:, 0:1] + deg2_ref[:, 0:1]
    dinv = lax.rsqrt(cnt + 1.0)
    dinv_ref[...] = jnp.broadcast_to(dinv, (_BLK, 128))
    xt = x_ref[...] * dinv
    for r, piece in zip(xt_refs, _split(xt, nout)):
      r[...] = piece

  return pl.pallas_call(
      body,
      grid=(nb,),
      in_specs=[_row_spec(f_in), _row_spec(16), _row_spec(16)],
      out_specs=[_row_spec(128)] + [_row_spec(q)] * nout,
      out_shape=[jax.ShapeDtypeStruct((n_pad, 128), jnp.float32)]
      + [jax.ShapeDtypeStruct((n_pad, q), jnp.float32)] * nout,
  )


@functools.lru_cache(maxsize=None)
def _make_tc01(n_pad, f_in, h_dim, nin, nout):
  nb = n_pad // _BLK
  qi = f_in // nin
  qo = h_dim // nout

  def body(*refs):
    it = iter(refs)
    accs = [next(it) for _ in range(nin)]
    xts = [next(it) for _ in range(nin)]
    dinv_ref, b_ref, w0_ref, w1_ref = next(it), next(it), next(it), next(it)
    outs = list(it)

    dinv = dinv_ref[:, 0:1]
    p = jnp.concatenate([dinv * (a[...] + t[...]) for a, t in zip(accs, xts)],
                        axis=1)
    h1 = jnp.maximum(_dot(p, w0_ref[...]) + b_ref[...], 0.0)
    for r, piece in zip(outs[:nout], _split(h1, nout)):
      r[...] = piece
    u1 = _dot(h1, w1_ref[...]) * dinv
    for r, piece in zip(outs[nout:], _split(u1, nout)):
      r[...] = piece

  in_specs = [_row_spec(qi)] * (2 * nin)
  in_specs += [_row_spec(128), _full_spec(1, h_dim),
               _full_spec(f_in, h_dim), _full_spec(h_dim, h_dim)]
  out_specs = [_row_spec(qo)] * (2 * nout)
  out_shape = [jax.ShapeDtypeStruct((n_pad, qo), jnp.float32)] * (2 * nout)

  return pl.pallas_call(
      body, grid=(nb,), in_specs=in_specs, out_specs=out_specs,
      out_shape=out_shape)


@functools.lru_cache(maxsize=None)
def _make_tc_layer(n_pad, h_dim, w_cols, nin, nout, with_resid, emit_h):
  nb = n_pad // _BLK
  qi = h_dim // nin
  qo = w_cols // nout

  def body(*refs):
    it = iter(refs)
    accs = [next(it) for _ in range(nin)]
    us = [next(it) for _ in range(nin)]
    rs = [next(it) for _ in range(nin)] if with_resid else None
    dinv_ref, b_ref, w_ref = next(it), next(it), next(it)
    outs = list(it)

    dinv = dinv_ref[:, 0:1]
    b = b_ref[...]
    hs = []
    for k in range(nin):
      hk = jnp.maximum(
          dinv * (accs[k][...] + us[k][...]) + b[:, k * qi:(k + 1) * qi], 0.0)
      if with_resid:
        hk = hk + rs[k][...]
      hs.append(hk)
    if emit_h:
      for r, hk in zip(outs[:nin], hs):
        r[...] = hk
      outs = outs[nin:]
    h = jnp.concatenate(hs, axis=1)
    u = _dot(h, w_ref[...]) * dinv
    for r, piece in zip(outs, _split(u, nout)):
      r[...] = piece

  in_specs = [_row_spec(qi)] * (nin * (3 if with_resid else 2))
  in_specs += [_row_spec(128), _full_spec(1, h_dim), _full_spec(h_dim, w_cols)]

  out_specs = []
  out_shape = []
  if emit_h:
    out_specs += [_row_spec(qi)] * nin
    out_shape += [jax.ShapeDtypeStruct((n_pad, qi), jnp.float32)] * nin
  out_specs += [_row_spec(qo)] * nout
  out_shape += [jax.ShapeDtypeStruct((n_pad, qo), jnp.float32)] * nout

  return pl.pallas_call(
      body, grid=(nb,), in_specs=in_specs, out_specs=out_specs,
      out_shape=out_shape)


@functools.lru_cache(maxsize=None)
def _make_tc_pool(n_pad, c_dim, nin):
  nb = n_pad // _BLK
  qi = c_dim // nin

  def body(*refs):
    it = iter(refs)
    accs = [next(it) for _ in range(nin)]
    us = [next(it) for _ in range(nin)]
    dinv_ref, b_ref, bt_ref, out_ref, pooled = (next(it), next(it), next(it),
                                                next(it), next(it))
    i = pl.program_id(0)

    @pl.when(i == 0)
    def _():
      pooled[...] = jnp.zeros((_G, 128), jnp.float32)

    dinv = dinv_ref[:, 0:1]
    h = dinv * jnp.concatenate([a[...] + u[...] for a, u in zip(accs, us)],
                               axis=1) + b_ref[...]
    xc = jnp.concatenate([h, jnp.ones((_BLK, 128 - c_dim), jnp.float32)],
                         axis=1)
    onehot = (bt_ref[...] == lax.broadcasted_iota(jnp.int32, (_BLK, _G), 1)
              ).astype(jnp.float32)
    pooled[...] += lax.dot_general(
        onehot, xc, (((0,), (0,)), ((), ())),
        preferred_element_type=jnp.float32)

    @pl.when(i == nb - 1)
    def _():
      p = pooled[...]
      mean = p[:, :c_dim] / jnp.maximum(p[:, c_dim:c_dim + 1], 1.0)
      z = mean - jnp.max(mean, axis=1, keepdims=True)
      lse = jnp.log(jnp.sum(jnp.exp(z), axis=1, keepdims=True))
      out_ref[...] = z - lse

  return pl.pallas_call(
      body,
      grid=(nb,),
      in_specs=[_row_spec(qi)] * (2 * nin)
      + [_row_spec(128), _full_spec(1, c_dim),
         pl.BlockSpec((_BLK, 1), lambda i: (i, 0))],
      out_specs=pl.BlockSpec((_G, c_dim), lambda i: (0, 0)),
      out_shape=jax.ShapeDtypeStruct((_G, c_dim), jnp.float32),
      scratch_shapes=[pltpu.VMEM((_G, 128), jnp.float32)],
  )


def kernel(x, edge_index, batch, W0, b0, W1, b1, W2, b2, W3, b3):
  n, f_in = x.shape
  e = edge_index.shape[1]
  h_dim = W0.shape[1]
  c_dim = W3.shape[1]

  n_pad = -(-n // 2048) * 2048
  nchunk = -(-e // (_NSUB * _CH))
  ep = _NSUB * nchunk * _CH

  pad_idx = jnp.full((ep - e,), n, jnp.int32)
  srcb = jnp.concatenate([edge_index[0].astype(jnp.int32), pad_idx]
                         ).reshape(_NSUB, nchunk, _CH)
  dstb = jnp.concatenate([edge_index[1].astype(jnp.int32), pad_idx]
                         ).reshape(_NSUB, nchunk, _CH)

  xp = jnp.pad(x, ((0, n_pad - n), (0, 0)))
  bt = jnp.pad(batch.astype(jnp.int32), (0, n_pad - n),
               constant_values=_G).reshape(n_pad, 1)

  ones16 = jnp.ones((_CH, 16), jnp.float32)
  z16 = jnp.zeros((_ZB, 16), jnp.float32)
  zh = jnp.zeros((_ZB, h_dim // 4), jnp.float32)
  zc = jnp.zeros((_ZB, c_dim // 2), jnp.float32)

  prop_deg = _make_propagate(n_pad, 16, nchunk, 1, const_rows=True)
  prop_x = _make_propagate(n_pad, f_in // 2, nchunk, 1)
  prop_h = _make_propagate(n_pad, h_dim // 4, nchunk, 2)
  prop_c = _make_propagate(n_pad, c_dim // 2, nchunk, 1)

  tc0 = _make_tc0(n_pad, f_in, 2)
  tc1 = _make_tc01(n_pad, f_in, h_dim, 2, 4)
  tc2 = _make_tc_layer(n_pad, h_dim, h_dim, 4, 4, True, False)
  tc3 = _make_tc_layer(n_pad, h_dim, c_dim, 4, 2, True, False)
  tc4 = _make_tc_pool(n_pad, c_dim, 2)

  b0r = b0.reshape(1, -1)
  b1r = b1.reshape(1, -1)
  b2r = b2.reshape(1, -1)
  b3r = b3.reshape(1, -1)

  zx = jnp.zeros((_ZB, f_in // 2), jnp.float32)
  deg0, deg1 = prop_deg(ones16, ones16, srcb, dstb, z16)
  dinv_b, *xt = tc0(xp, deg0, deg1)

  ax = prop_x(*xt, srcb, dstb, zx)
  h1a, h1b, h1c, h1d, *u1 = tc1(*ax, *xt, dinv_b, b0r, W0, W1)

  a1 = prop_h(*u1, srcb, dstb, zh)
  u2 = tc2(*a1, *u1, h1a, h1b, h1c, h1d, dinv_b, b1r, W2)

  a2 = prop_h(*u2, srcb, dstb, zh)
  u3 = tc3(*a2, *u2, h1a, h1b, h1c, h1d, dinv_b, b2r, W3)

  a3 = prop_c(*u3, srcb, dstb, zc)
  return tc4(*a3, *u3, dinv_b, b3r, bt)
